# baseline (speedup 1.0000x reference)
import functools

import jax
import jax.numpy as jnp
from jax import lax
from jax.experimental import pallas as pl
from jax.experimental.pallas import tpu as pltpu
from jax.experimental.pallas import tpu_sc as plsc
from jax._src.pallas import mpmd as _pl_mpmd

B = 4096
N_HIST = 20
CAP_LEN = 50
TOK = N_HIST + CAP_LEN
ACT_VOCAB = 1000
CAP_VOCAB = 100000
DIM = 128

_CAP_BLK = 4000

_NC = 2
_NS = 16
_NW = _NC * _NS
_BPW = B // _NW
_NBUF = 6
_DIST = 3


def _proj_block(x_ref, w_ref, b_ref, o_ref):
    o_ref[...] = (
        jnp.dot(x_ref[...], w_ref[...], preferred_element_type=jnp.float32)
        + b_ref[...]
    )


def _project_actions(action_emb, Wa, ba):
    return pl.pallas_call(
        _proj_block,
        out_shape=jax.ShapeDtypeStruct((ACT_VOCAB, DIM), jnp.float32),
    )(action_emb, Wa, ba.reshape(1, DIM))


def _project_captions(caption_emb, Wc, bc):
    n_blocks = CAP_VOCAB // _CAP_BLK
    return pl.pallas_call(
        _proj_block,
        grid=(n_blocks,),
        in_specs=[
            pl.BlockSpec((_CAP_BLK, DIM), lambda i: (i, 0)),
            pl.BlockSpec((DIM, DIM), lambda i: (0, 0)),
            pl.BlockSpec((1, DIM), lambda i: (0, 0)),
        ],
        out_specs=pl.BlockSpec((_CAP_BLK, DIM), lambda i: (i, 0)),
        out_shape=jax.ShapeDtypeStruct((CAP_VOCAB, DIM), jnp.float32),
    )(caption_emb, Wc, bc.reshape(1, DIM))


def _ring_gather(tbl, idx_v, n, tbase, out_hbm, bufs, gsem, osem, cb):
    dist = min(_DIST, n)
    for t in range(dist):
        pltpu.make_async_copy(
            tbl.at[idx_v.at[t]], bufs.at[t % _NBUF], gsem
        ).start()

    def body(i, carry):
        @pl.when(i >= dist)
        def _():
            pltpu.make_async_copy(
                bufs.at[(i - dist) % _NBUF],
                out_hbm.at[tbase + i - dist, pl.ds(cb, _BPW)],
                osem,
            ).wait()

        @pl.when(i < n - dist)
        def _():
            pltpu.make_async_copy(
                tbl.at[idx_v.at[i + dist]],
                bufs.at[(i + dist) % _NBUF],
                gsem,
            ).start()

        pltpu.make_async_copy(
            tbl.at[idx_v.at[i]], bufs.at[i % _NBUF], gsem
        ).wait()
        pltpu.make_async_copy(
            bufs.at[i % _NBUF], out_hbm.at[tbase + i, pl.ds(cb, _BPW)], osem
        ).start()
        return carry

    lax.fori_loop(0, n, body, 0)

    for t in range(n - dist, n):
        pltpu.make_async_copy(
            bufs.at[t % _NBUF], out_hbm.at[tbase + t, pl.ds(cb, _BPW)], osem
        ).wait()


def _transpose_idx(raw_v, idx_v, n_tok):
    lanes = jax.lax.iota(jnp.int32, 16)

    def body(t, carry):
        for g in range(_BPW // 16):
            flat = (g * 16 + lanes) * n_tok + t
            vals = plsc.load_gather(raw_v, [flat])
            idx_v[t, pl.ds(g * 16, 16)] = vals
        return carry

    lax.fori_loop(0, n_tok, body, 0)


def _gather_actions(pa, actions):
    mesh = plsc.VectorSubcoreMesh(core_axis_name="c", subcore_axis_name="s")

    @functools.partial(
        pl.kernel,
        mesh=mesh,
        out_type=jax.ShapeDtypeStruct((TOK, B, DIM), jnp.float32),
        compiler_params=pltpu.CompilerParams(needs_layout_passes=False),
        scratch_types=[
            pltpu.VMEM((_BPW * N_HIST,), jnp.int32),
            pltpu.VMEM((N_HIST, _BPW), jnp.int32),
            pltpu.VMEM((_NBUF, _BPW, DIM), jnp.float32),
            pltpu.SemaphoreType.DMA,
            pltpu.SemaphoreType.DMA,
        ],
    )
    def k(pa_hbm, act_hbm, out_hbm, raw_v, ia_v, bufs, gsem, osem):
        wid = lax.axis_index("s") * _NC + lax.axis_index("c")
        cb = wid * _BPW
        pltpu.sync_copy(act_hbm.at[pl.ds(cb * N_HIST, _BPW * N_HIST)], raw_v)
        _transpose_idx(raw_v, ia_v, N_HIST)
        _ring_gather(pa_hbm, ia_v, N_HIST, 0, out_hbm, bufs, gsem, osem, cb)

    return k(pa, actions)


def _gather_captions(pc, captions, prev):
    mesh = plsc.VectorSubcoreMesh(core_axis_name="c", subcore_axis_name="s")

    def k(pc_hbm, cap_hbm, prev_hbm, out_hbm, raw_v, ic_v, bufs, gsem, osem):
        del prev_hbm
        wid = lax.axis_index("s") * _NC + lax.axis_index("c")
        cb = wid * _BPW
        pltpu.sync_copy(cap_hbm.at[pl.ds(cb * CAP_LEN, _BPW * CAP_LEN)], raw_v)
        _transpose_idx(raw_v, ic_v, CAP_LEN)
        _ring_gather(pc_hbm, ic_v, CAP_LEN, N_HIST, out_hbm, bufs, gsem, osem, cb)

    f = _pl_mpmd._mpmd_map(
        [(mesh, k)],
        jax.ShapeDtypeStruct((TOK, B, DIM), jnp.float32),
        input_output_aliases={2: 0},
        compiler_params=pltpu.CompilerParams(needs_layout_passes=False),
        scratch_types=[
            pltpu.VMEM((_BPW * CAP_LEN,), jnp.int32),
            pltpu.VMEM((CAP_LEN, _BPW), jnp.int32),
            pltpu.VMEM((_NBUF, _BPW, DIM), jnp.float32),
            pltpu.SemaphoreType.DMA,
            pltpu.SemaphoreType.DMA,
        ],
    )
    return f(pc, captions, prev)


def kernel(actions, captions, action_emb, Wa, ba, caption_emb, Wc, bc):
    pa = _project_actions(action_emb, Wa, ba)
    pc = _project_captions(caption_emb, Wc, bc)
    out_a = _gather_actions(pa, actions.reshape(-1))
    out_t = _gather_captions(pc, captions.reshape(-1), out_a)
    enc = out_t.transpose(1, 0, 2)
    obs_mask = jnp.concatenate(
        [jnp.zeros((B, N_HIST), dtype=bool), jnp.ones((B, CAP_LEN), dtype=bool)],
        axis=1,
    )
    return enc, obs_mask


# speedup vs baseline: 1.0577x; 1.0577x over previous
import functools

import jax
import jax.numpy as jnp
from jax import lax
from jax.experimental import pallas as pl
from jax.experimental.pallas import tpu as pltpu
from jax.experimental.pallas import tpu_sc as plsc
from jax._src.pallas import mpmd as _pl_mpmd

B = 4096
N_HIST = 20
CAP_LEN = 50
TOK = N_HIST + CAP_LEN
ACT_VOCAB = 1000
CAP_VOCAB = 100000
DIM = 128

_CAP_BLK = 4000

_NC = 2
_NS = 16
_NW = _NC * _NS
_BPW = B // _NW
_NBUF = 6
_DIST = 3


def _proj_block(x_ref, w_ref, b_ref, o_ref):
    o_ref[...] = (
        jnp.dot(x_ref[...], w_ref[...], preferred_element_type=jnp.float32)
        + b_ref[...]
    )


def _project_actions(action_emb, Wa, ba):
    return pl.pallas_call(
        _proj_block,
        out_shape=jax.ShapeDtypeStruct((ACT_VOCAB, DIM), jnp.float32),
    )(action_emb, Wa, ba.reshape(1, DIM))


def _project_captions(caption_emb, Wc, bc):
    n_blocks = CAP_VOCAB // _CAP_BLK
    return pl.pallas_call(
        _proj_block,
        grid=(n_blocks,),
        in_specs=[
            pl.BlockSpec((_CAP_BLK, DIM), lambda i: (i, 0)),
            pl.BlockSpec((DIM, DIM), lambda i: (0, 0)),
            pl.BlockSpec((1, DIM), lambda i: (0, 0)),
        ],
        out_specs=pl.BlockSpec((_CAP_BLK, DIM), lambda i: (i, 0)),
        out_shape=jax.ShapeDtypeStruct((CAP_VOCAB, DIM), jnp.float32),
    )(caption_emb, Wc, bc.reshape(1, DIM))


def _ring_gather(tbl, idx_v, n, tbase, out_hbm, bufs, gsem, osem, cb):
    dist = min(_DIST, n)
    for t in range(dist):
        pltpu.make_async_copy(
            tbl.at[idx_v.at[t]], bufs.at[t % _NBUF], gsem
        ).start()

    def body(i, carry):
        @pl.when(i >= dist)
        def _():
            pltpu.make_async_copy(
                bufs.at[(i - dist) % _NBUF],
                out_hbm.at[tbase + i - dist, pl.ds(cb, _BPW)],
                osem,
            ).wait()

        @pl.when(i < n - dist)
        def _():
            pltpu.make_async_copy(
                tbl.at[idx_v.at[i + dist]],
                bufs.at[(i + dist) % _NBUF],
                gsem,
            ).start()

        pltpu.make_async_copy(
            tbl.at[idx_v.at[i]], bufs.at[i % _NBUF], gsem
        ).wait()
        pltpu.make_async_copy(
            bufs.at[i % _NBUF], out_hbm.at[tbase + i, pl.ds(cb, _BPW)], osem
        ).start()
        return carry

    lax.fori_loop(0, n, body, 0)

    for t in range(n - dist, n):
        pltpu.make_async_copy(
            bufs.at[t % _NBUF], out_hbm.at[tbase + t, pl.ds(cb, _BPW)], osem
        ).wait()


def _gather_actions(pa, ia):
    mesh = plsc.VectorSubcoreMesh(core_axis_name="c", subcore_axis_name="s")

    @functools.partial(
        pl.kernel,
        mesh=mesh,
        out_type=jax.ShapeDtypeStruct((TOK, B, DIM), jnp.float32),
        scratch_types=[
            pltpu.VMEM((N_HIST, _BPW), jnp.int32),
            pltpu.VMEM((_NBUF, _BPW, DIM), jnp.float32),
            pltpu.SemaphoreType.DMA,
            pltpu.SemaphoreType.DMA,
        ],
    )
    def k(pa_hbm, ia_hbm, out_hbm, ia_v, bufs, gsem, osem):
        wid = lax.axis_index("s") * _NC + lax.axis_index("c")
        cb = wid * _BPW
        pltpu.sync_copy(ia_hbm.at[wid], ia_v)
        _ring_gather(pa_hbm, ia_v, N_HIST, 0, out_hbm, bufs, gsem, osem, cb)

    return k(pa, ia)


def _gather_captions(pc, ic, prev):
    mesh = plsc.VectorSubcoreMesh(core_axis_name="c", subcore_axis_name="s")

    def k(pc_hbm, ic_hbm, prev_hbm, out_hbm, ic_v, bufs, gsem, osem):
        del prev_hbm
        wid = lax.axis_index("s") * _NC + lax.axis_index("c")
        cb = wid * _BPW
        pltpu.sync_copy(ic_hbm.at[wid], ic_v)
        _ring_gather(pc_hbm, ic_v, CAP_LEN, N_HIST, out_hbm, bufs, gsem, osem, cb)

    f = _pl_mpmd._mpmd_map(
        [(mesh, k)],
        jax.ShapeDtypeStruct((TOK, B, DIM), jnp.float32),
        input_output_aliases={2: 0},
        scratch_types=[
            pltpu.VMEM((CAP_LEN, _BPW), jnp.int32),
            pltpu.VMEM((_NBUF, _BPW, DIM), jnp.float32),
            pltpu.SemaphoreType.DMA,
            pltpu.SemaphoreType.DMA,
        ],
    )
    return f(pc, ic, prev)


def kernel(actions, captions, action_emb, Wa, ba, caption_emb, Wc, bc):
    pa = _project_actions(action_emb, Wa, ba)
    pc = _project_captions(caption_emb, Wc, bc)
    ia = actions.reshape(_NW, _BPW, N_HIST).transpose(0, 2, 1)
    ic = captions.reshape(_NW, _BPW, CAP_LEN).transpose(0, 2, 1)
    out_a = _gather_actions(pa, ia)
    out_t = _gather_captions(pc, ic, out_a)
    enc = out_t.transpose(1, 0, 2)
    obs_mask = jnp.concatenate(
        [jnp.zeros((B, N_HIST), dtype=bool), jnp.ones((B, CAP_LEN), dtype=bool)],
        axis=1,
    )
    return enc, obs_mask


# speedup vs baseline: 1.0667x; 1.0086x over previous
import functools

import jax
import jax.numpy as jnp
from jax import lax
from jax.experimental import pallas as pl
from jax.experimental.pallas import tpu as pltpu
from jax.experimental.pallas import tpu_sc as plsc
from jax._src.pallas import mpmd as _pl_mpmd

B = 4096
N_HIST = 20
CAP_LEN = 50
TOK = N_HIST + CAP_LEN
ACT_VOCAB = 1000
CAP_VOCAB = 100000
DIM = 128

_CAP_BLK = 10000

_NC = 2
_NS = 16
_NW = _NC * _NS
_BPW = B // _NW
_NBUF = 6
_DIST = 3


def _proj_block(x_ref, w_ref, b_ref, o_ref):
    o_ref[...] = (
        jnp.dot(x_ref[...], w_ref[...], preferred_element_type=jnp.float32)
        + b_ref[...]
    )


def _project_actions(action_emb, Wa, ba):
    return pl.pallas_call(
        _proj_block,
        out_shape=jax.ShapeDtypeStruct((ACT_VOCAB, DIM), jnp.float32),
    )(action_emb, Wa, ba.reshape(1, DIM))


def _project_captions(caption_emb, Wc, bc):
    n_blocks = CAP_VOCAB // _CAP_BLK
    return pl.pallas_call(
        _proj_block,
        grid=(n_blocks,),
        in_specs=[
            pl.BlockSpec((_CAP_BLK, DIM), lambda i: (i, 0)),
            pl.BlockSpec((DIM, DIM), lambda i: (0, 0)),
            pl.BlockSpec((1, DIM), lambda i: (0, 0)),
        ],
        out_specs=pl.BlockSpec((_CAP_BLK, DIM), lambda i: (i, 0)),
        out_shape=jax.ShapeDtypeStruct((CAP_VOCAB, DIM), jnp.float32),
    )(caption_emb, Wc, bc.reshape(1, DIM))


def _ring_gather(tbl, idx_v, n, tbase, out_hbm, bufs, gsem, osem, cb):
    dist = min(_DIST, n)
    for t in range(dist):
        pltpu.make_async_copy(
            tbl.at[idx_v.at[t]], bufs.at[t % _NBUF], gsem
        ).start()

    def body(i, carry):
        @pl.when(i >= dist)
        def _():
            pltpu.make_async_copy(
                bufs.at[(i - dist) % _NBUF],
                out_hbm.at[tbase + i - dist, pl.ds(cb, _BPW)],
                osem,
            ).wait()

        @pl.when(i < n - dist)
        def _():
            pltpu.make_async_copy(
                tbl.at[idx_v.at[i + dist]],
                bufs.at[(i + dist) % _NBUF],
                gsem,
            ).start()

        pltpu.make_async_copy(
            tbl.at[idx_v.at[i]], bufs.at[i % _NBUF], gsem
        ).wait()
        pltpu.make_async_copy(
            bufs.at[i % _NBUF], out_hbm.at[tbase + i, pl.ds(cb, _BPW)], osem
        ).start()
        return carry

    lax.fori_loop(0, n, body, 0)

    for t in range(n - dist, n):
        pltpu.make_async_copy(
            bufs.at[t % _NBUF], out_hbm.at[tbase + t, pl.ds(cb, _BPW)], osem
        ).wait()


def _gather_actions(pa, ia):
    mesh = plsc.VectorSubcoreMesh(core_axis_name="c", subcore_axis_name="s")

    @functools.partial(
        pl.kernel,
        mesh=mesh,
        out_type=jax.ShapeDtypeStruct((TOK, B, DIM), jnp.float32),
        scratch_types=[
            pltpu.VMEM((N_HIST, _BPW), jnp.int32),
            pltpu.VMEM((_NBUF, _BPW, DIM), jnp.float32),
            pltpu.SemaphoreType.DMA,
            pltpu.SemaphoreType.DMA,
        ],
    )
    def k(pa_hbm, ia_hbm, out_hbm, ia_v, bufs, gsem, osem):
        wid = lax.axis_index("s") * _NC + lax.axis_index("c")
        cb = wid * _BPW
        pltpu.sync_copy(ia_hbm.at[wid], ia_v)
        _ring_gather(pa_hbm, ia_v, N_HIST, 0, out_hbm, bufs, gsem, osem, cb)

    return k(pa, ia)


def _gather_captions(pc, ic, prev):
    mesh = plsc.VectorSubcoreMesh(core_axis_name="c", subcore_axis_name="s")

    def k(pc_hbm, ic_hbm, prev_hbm, out_hbm, ic_v, bufs, gsem, osem):
        del prev_hbm
        wid = lax.axis_index("s") * _NC + lax.axis_index("c")
        cb = wid * _BPW
        pltpu.sync_copy(ic_hbm.at[wid], ic_v)
        _ring_gather(pc_hbm, ic_v, CAP_LEN, N_HIST, out_hbm, bufs, gsem, osem, cb)

    f = _pl_mpmd._mpmd_map(
        [(mesh, k)],
        jax.ShapeDtypeStruct((TOK, B, DIM), jnp.float32),
        input_output_aliases={2: 0},
        scratch_types=[
            pltpu.VMEM((CAP_LEN, _BPW), jnp.int32),
            pltpu.VMEM((_NBUF, _BPW, DIM), jnp.float32),
            pltpu.SemaphoreType.DMA,
            pltpu.SemaphoreType.DMA,
        ],
    )
    return f(pc, ic, prev)


def kernel(actions, captions, action_emb, Wa, ba, caption_emb, Wc, bc):
    pa = _project_actions(action_emb, Wa, ba)
    pc = _project_captions(caption_emb, Wc, bc)
    ia = actions.reshape(_NW, _BPW, N_HIST).transpose(0, 2, 1)
    ic = captions.reshape(_NW, _BPW, CAP_LEN).transpose(0, 2, 1)
    out_a = _gather_actions(pa, ia)
    out_t = _gather_captions(pc, ic, out_a)
    enc = out_t.transpose(1, 0, 2)
    obs_mask = jnp.concatenate(
        [jnp.zeros((B, N_HIST), dtype=bool), jnp.ones((B, CAP_LEN), dtype=bool)],
        axis=1,
    )
    return enc, obs_mask


# speedup vs baseline: 1.0760x; 1.0087x over previous
import functools

import jax
import jax.numpy as jnp
from jax import lax
from jax.experimental import pallas as pl
from jax.experimental.pallas import tpu as pltpu
from jax.experimental.pallas import tpu_sc as plsc
from jax._src.pallas import mpmd as _pl_mpmd

B = 4096
N_HIST = 20
CAP_LEN = 50
TOK = N_HIST + CAP_LEN
ACT_VOCAB = 1000
CAP_VOCAB = 100000
DIM = 128

_CAP_BLK = 20000

_NC = 2
_NS = 16
_NW = _NC * _NS
_BPW = B // _NW
_NBUF = 6
_DIST = 3


def _proj_block(x_ref, w_ref, b_ref, o_ref):
    o_ref[...] = (
        jnp.dot(x_ref[...], w_ref[...], preferred_element_type=jnp.float32)
        + b_ref[...]
    )


def _project_actions(action_emb, Wa, ba):
    return pl.pallas_call(
        _proj_block,
        out_shape=jax.ShapeDtypeStruct((ACT_VOCAB, DIM), jnp.float32),
    )(action_emb, Wa, ba.reshape(1, DIM))


def _project_captions(caption_emb, Wc, bc):
    n_blocks = CAP_VOCAB // _CAP_BLK
    return pl.pallas_call(
        _proj_block,
        grid=(n_blocks,),
        in_specs=[
            pl.BlockSpec((_CAP_BLK, DIM), lambda i: (i, 0)),
            pl.BlockSpec((DIM, DIM), lambda i: (0, 0)),
            pl.BlockSpec((1, DIM), lambda i: (0, 0)),
        ],
        out_specs=pl.BlockSpec((_CAP_BLK, DIM), lambda i: (i, 0)),
        out_shape=jax.ShapeDtypeStruct((CAP_VOCAB, DIM), jnp.float32),
    )(caption_emb, Wc, bc.reshape(1, DIM))


def _ring_gather(tbl, idx_v, n, tbase, out_hbm, bufs, gsem, osem, cb):
    dist = min(_DIST, n)
    for t in range(dist):
        pltpu.make_async_copy(
            tbl.at[idx_v.at[t]], bufs.at[t % _NBUF], gsem
        ).start()

    def body(i, carry):
        @pl.when(i >= dist)
        def _():
            pltpu.make_async_copy(
                bufs.at[(i - dist) % _NBUF],
                out_hbm.at[tbase + i - dist, pl.ds(cb, _BPW)],
                osem,
            ).wait()

        @pl.when(i < n - dist)
        def _():
            pltpu.make_async_copy(
                tbl.at[idx_v.at[i + dist]],
                bufs.at[(i + dist) % _NBUF],
                gsem,
            ).start()

        pltpu.make_async_copy(
            tbl.at[idx_v.at[i]], bufs.at[i % _NBUF], gsem
        ).wait()
        pltpu.make_async_copy(
            bufs.at[i % _NBUF], out_hbm.at[tbase + i, pl.ds(cb, _BPW)], osem
        ).start()
        return carry

    lax.fori_loop(0, n, body, 0)

    for t in range(n - dist, n):
        pltpu.make_async_copy(
            bufs.at[t % _NBUF], out_hbm.at[tbase + t, pl.ds(cb, _BPW)], osem
        ).wait()


def _gather_actions(pa, ia):
    mesh = plsc.VectorSubcoreMesh(core_axis_name="c", subcore_axis_name="s")

    @functools.partial(
        pl.kernel,
        mesh=mesh,
        out_type=jax.ShapeDtypeStruct((TOK, B, DIM), jnp.float32),
        scratch_types=[
            pltpu.VMEM((N_HIST, _BPW), jnp.int32),
            pltpu.VMEM((_NBUF, _BPW, DIM), jnp.float32),
            pltpu.SemaphoreType.DMA,
            pltpu.SemaphoreType.DMA,
        ],
    )
    def k(pa_hbm, ia_hbm, out_hbm, ia_v, bufs, gsem, osem):
        wid = lax.axis_index("s") * _NC + lax.axis_index("c")
        cb = wid * _BPW
        pltpu.sync_copy(ia_hbm.at[wid], ia_v)
        _ring_gather(pa_hbm, ia_v, N_HIST, 0, out_hbm, bufs, gsem, osem, cb)

    return k(pa, ia)


def _gather_captions(pc, ic, prev):
    mesh = plsc.VectorSubcoreMesh(core_axis_name="c", subcore_axis_name="s")

    def k(pc_hbm, ic_hbm, prev_hbm, out_hbm, ic_v, bufs, gsem, osem):
        del prev_hbm
        wid = lax.axis_index("s") * _NC + lax.axis_index("c")
        cb = wid * _BPW
        pltpu.sync_copy(ic_hbm.at[wid], ic_v)
        _ring_gather(pc_hbm, ic_v, CAP_LEN, N_HIST, out_hbm, bufs, gsem, osem, cb)

    f = _pl_mpmd._mpmd_map(
        [(mesh, k)],
        jax.ShapeDtypeStruct((TOK, B, DIM), jnp.float32),
        input_output_aliases={2: 0},
        scratch_types=[
            pltpu.VMEM((CAP_LEN, _BPW), jnp.int32),
            pltpu.VMEM((_NBUF, _BPW, DIM), jnp.float32),
            pltpu.SemaphoreType.DMA,
            pltpu.SemaphoreType.DMA,
        ],
    )
    return f(pc, ic, prev)


def kernel(actions, captions, action_emb, Wa, ba, caption_emb, Wc, bc):
    pa = _project_actions(action_emb, Wa, ba)
    pc = _project_captions(caption_emb, Wc, bc)
    ia = actions.reshape(_NW, _BPW, N_HIST).transpose(0, 2, 1)
    ic = captions.reshape(_NW, _BPW, CAP_LEN).transpose(0, 2, 1)
    out_a = _gather_actions(pa, ia)
    out_t = _gather_captions(pc, ic, out_a)
    enc = out_t.transpose(1, 0, 2)
    obs_mask = jnp.concatenate(
        [jnp.zeros((B, N_HIST), dtype=bool), jnp.ones((B, CAP_LEN), dtype=bool)],
        axis=1,
    )
    return enc, obs_mask
